# Initial kernel scaffold; baseline (speedup 1.0000x reference)
import jax
import jax.numpy as jnp
from jax.experimental import pallas as pl


def kernel(x, edge_index, W1, b1, W2, b2, Wl, bl):
    raise NotImplementedError("write your pallas kernel here")


# speedup vs baseline: 5.9479x; 5.9479x over previous
import functools

import jax
import jax.numpy as jnp
from jax import lax
from jax.experimental import pallas as pl
from jax.experimental.pallas import tpu as pltpu
from jax.experimental.pallas import tpu_sc as plsc

N = 10000
E = 320000
D = 128
NC = 2
NS = 16
NW = NC * NS
CH = 128
NCHUNK = E // CH
ITERS = -(-NCHUNK // NW)
RPT = N // NS

_mesh = plsc.VectorSubcoreMesh(
    core_axis_name="c", subcore_axis_name="s", num_cores=NC, num_subcores=NS)


def _zero_vmem_2d(ref, rows):
    def body(i, carry):
        r = i // (D // 16)
        col = (i % (D // 16)) * 16
        ref[r, pl.ds(col, 16)] = jnp.zeros((16,), jnp.float32)
        return carry
    lax.fori_loop(0, rows * (D // 16), body, 0)


def _sc_degrees_body(src_hbm, dst_hbm, out_hbm, sidx, didx, ones, zbuf,
                     dsp_out, dsp_in):
    c = lax.axis_index("c")
    s = lax.axis_index("s")
    w = s * NC + c

    def fill_ones(i, carry):
        ones[pl.ds(i * 16, 16)] = jnp.full((16,), 1.0, jnp.float32)
        return carry
    lax.fori_loop(0, CH // 16, fill_ones, 0)

    def fill_zeros(i, carry):
        zbuf[pl.ds(i * 16, 16)] = jnp.zeros((16,), jnp.float32)
        return carry
    lax.fori_loop(0, 64, fill_zeros, 0)

    @pl.when(s < 10)
    def _():
        pltpu.sync_copy(zbuf.at[pl.ds(0, 1000)], dsp_out.at[pl.ds(s * 1000, 1000)])
        pltpu.sync_copy(zbuf.at[pl.ds(0, 1000)], dsp_in.at[pl.ds(s * 1000, 1000)])
    plsc.subcore_barrier()

    def step(i, carry):
        cid = w + i * NW
        @pl.when(cid < NCHUNK)
        def _():
            base = cid * CH
            pltpu.sync_copy(src_hbm.at[pl.ds(base, CH)], sidx)
            pltpu.sync_copy(dst_hbm.at[pl.ds(base, CH)], didx)
            pltpu.sync_copy(ones, dsp_out.at[sidx], add=True)
            pltpu.sync_copy(ones, dsp_in.at[didx], add=True)
        return carry
    lax.fori_loop(0, ITERS, step, 0)
    plsc.subcore_barrier()

    @pl.when(s < 10)
    def _():
        pltpu.sync_copy(dsp_out.at[pl.ds(s * 1000, 1000)], zbuf.at[pl.ds(0, 1000)])
        pltpu.sync_copy(zbuf.at[pl.ds(0, 1000)],
                        out_hbm.at[pl.ds(c * 2 * N + s * 1000, 1000)])
        pltpu.sync_copy(dsp_in.at[pl.ds(s * 1000, 1000)], zbuf.at[pl.ds(0, 1000)])
        pltpu.sync_copy(zbuf.at[pl.ds(0, 1000)],
                        out_hbm.at[pl.ds(c * 2 * N + N + s * 1000, 1000)])


def _make_degrees(interpret=False):
    return pl.kernel(
        _sc_degrees_body,
        out_type=jax.ShapeDtypeStruct((NC * 2 * N,), jnp.float32),
        mesh=_mesh,
        scratch_types=[
            pltpu.VMEM((CH,), jnp.int32),
            pltpu.VMEM((CH,), jnp.int32),
            pltpu.VMEM((CH,), jnp.float32),
            pltpu.VMEM((1024,), jnp.float32),
            pltpu.VMEM_SHARED((N,), jnp.float32),
            pltpu.VMEM_SHARED((N,), jnp.float32),
        ],
        interpret=interpret,
    )


_sc_degrees = _make_degrees()


def _sc_scatter_body(g_hbm, src_hbm, dst_hbm, out_hbm, sidx, didx, stage, zbuf,
                     accum, sem):
    c = lax.axis_index("c")
    s = lax.axis_index("s")
    w = s * NC + c

    _zero_vmem_2d(zbuf, 200)
    for j in range(4):
        cidx = s + j * NS
        @pl.when(cidx < 50)
        def _():
            pltpu.sync_copy(zbuf, accum.at[pl.ds(cidx * 200, 200)])
    plsc.subcore_barrier()

    def step(i, carry):
        cid = w + i * NW
        @pl.when(cid < NCHUNK)
        def _():
            base = cid * CH
            pltpu.sync_copy(src_hbm.at[pl.ds(base, CH)], sidx)
            pltpu.sync_copy(dst_hbm.at[pl.ds(base, CH)], didx)
            pltpu.async_copy(g_hbm.at[sidx], stage, sem).wait()
            pltpu.sync_copy(stage, accum.at[didx], add=True)
        return carry
    lax.fori_loop(0, ITERS, step, 0)
    plsc.subcore_barrier()

    for j in range(4):
        cidx = s + j * NS
        @pl.when(cidx < 50)
        def _():
            rr = cidx * 200
            pltpu.sync_copy(accum.at[pl.ds(rr, 200)], zbuf)
            pltpu.sync_copy(zbuf, out_hbm.at[c, pl.ds(rr, 200)])


def _make_scatter(interpret=False):
    return pl.kernel(
        _sc_scatter_body,
        out_type=jax.ShapeDtypeStruct((NC, N, D), jnp.float32),
        mesh=_mesh,
        scratch_types=[
            pltpu.VMEM((CH,), jnp.int32),
            pltpu.VMEM((CH,), jnp.int32),
            pltpu.VMEM((CH, D), jnp.float32),
            pltpu.VMEM((200, D), jnp.float32),
            pltpu.VMEM_SHARED((N, D), jnp.float32),
            pltpu.SemaphoreType.DMA,
        ],
        interpret=interpret,
    )


_sc_scatter = _make_scatter()


_BLK = 1000
_GRID = N // _BLK


def _stage1_body(x_ref, do0, do1, di0, di1, w1_ref, g_ref, no_ref, ni_ref):
    deg_o = do0[...] + do1[...]
    deg_i = di0[...] + di1[...]
    n_out = lax.rsqrt(jnp.maximum(deg_o, 1.0))
    n_in = lax.rsqrt(jnp.maximum(deg_i, 1.0))
    h = x_ref[...] * n_out
    g_ref[...] = jnp.dot(h, w1_ref[...], preferred_element_type=jnp.float32)
    no_ref[...] = n_out
    ni_ref[...] = n_in


_stage1 = pl.pallas_call(
    _stage1_body,
    grid=(_GRID,),
    in_specs=[
        pl.BlockSpec((_BLK, D), lambda i: (i, 0)),
        pl.BlockSpec((_BLK, 1), lambda i: (i, 0)),
        pl.BlockSpec((_BLK, 1), lambda i: (i, 0)),
        pl.BlockSpec((_BLK, 1), lambda i: (i, 0)),
        pl.BlockSpec((_BLK, 1), lambda i: (i, 0)),
        pl.BlockSpec((D, D), lambda i: (0, 0)),
    ],
    out_specs=[
        pl.BlockSpec((_BLK, D), lambda i: (i, 0)),
        pl.BlockSpec((_BLK, 1), lambda i: (i, 0)),
        pl.BlockSpec((_BLK, 1), lambda i: (i, 0)),
    ],
    out_shape=[
        jax.ShapeDtypeStruct((N, D), jnp.float32),
        jax.ShapeDtypeStruct((N, 1), jnp.float32),
        jax.ShapeDtypeStruct((N, 1), jnp.float32),
    ],
)


def _stage2_body(aggp_ref, ni_ref, no_ref, b_ref, w_ref, g_ref):
    agg = aggp_ref[0] + aggp_ref[1]
    h = jnp.maximum(agg * ni_ref[...] + b_ref[...], 0.0)
    g_ref[...] = jnp.dot(h * no_ref[...], w_ref[...],
                         preferred_element_type=jnp.float32)


_stage2 = pl.pallas_call(
    _stage2_body,
    grid=(_GRID,),
    in_specs=[
        pl.BlockSpec((NC, _BLK, D), lambda i: (0, i, 0)),
        pl.BlockSpec((_BLK, 1), lambda i: (i, 0)),
        pl.BlockSpec((_BLK, 1), lambda i: (i, 0)),
        pl.BlockSpec((1, D), lambda i: (0, 0)),
        pl.BlockSpec((D, D), lambda i: (0, 0)),
    ],
    out_specs=pl.BlockSpec((_BLK, D), lambda i: (i, 0)),
    out_shape=jax.ShapeDtypeStruct((N, D), jnp.float32),
)


def _stage3_body(aggp_ref, ni_ref, b_ref, wl_ref, bl_ref, out_ref, acc_ref):
    i = pl.program_id(0)

    @pl.when(i == 0)
    def _():
        acc_ref[...] = jnp.zeros_like(acc_ref)

    agg = aggp_ref[0] + aggp_ref[1]
    h = jnp.maximum(agg * ni_ref[...] + b_ref[...], 0.0)
    acc_ref[...] += jnp.sum(h, axis=0, keepdims=True)

    @pl.when(i == _GRID - 1)
    def _():
        pooled = acc_ref[...] / jnp.float32(N)
        out_ref[...] = jnp.dot(pooled, wl_ref[...],
                               preferred_element_type=jnp.float32) + bl_ref[...]


_stage3 = pl.pallas_call(
    _stage3_body,
    grid=(_GRID,),
    in_specs=[
        pl.BlockSpec((NC, _BLK, D), lambda i: (0, i, 0)),
        pl.BlockSpec((_BLK, 1), lambda i: (i, 0)),
        pl.BlockSpec((1, D), lambda i: (0, 0)),
        pl.BlockSpec((D, 1), lambda i: (0, 0)),
        pl.BlockSpec((1, 1), lambda i: (0, 0)),
    ],
    out_specs=pl.BlockSpec((1, 1), lambda i: (0, 0)),
    out_shape=jax.ShapeDtypeStruct((1, 1), jnp.float32),
    scratch_shapes=[pltpu.VMEM((1, D), jnp.float32)],
)


def kernel(x, edge_index, W1, b1, W2, b2, Wl, bl):
    src = edge_index[0].astype(jnp.int32)
    dst = edge_index[1].astype(jnp.int32)

    degp = _sc_degrees(src, dst).reshape(NC, 2, N)
    do0 = degp[0, 0].reshape(N, 1)
    do1 = degp[1, 0].reshape(N, 1)
    di0 = degp[0, 1].reshape(N, 1)
    di1 = degp[1, 1].reshape(N, 1)

    g1, n_out, n_in = _stage1(x, do0, do1, di0, di1, W1)
    agg1p = _sc_scatter(g1, src, dst)
    g2 = _stage2(agg1p, n_in, n_out, b1.reshape(1, D), W2)
    agg2p = _sc_scatter(g2, src, dst)
    out = _stage3(agg2p, n_in, b2.reshape(1, D), Wl, bl.reshape(1, 1))
    return out
